# Initial kernel scaffold; baseline (speedup 1.0000x reference)
import jax
import jax.numpy as jnp
from jax.experimental import pallas as pl


def kernel(ceps, spec, vals_f, vals_t, rows_f, cols_f, rows_t, cols_t):
    raise NotImplementedError("write your pallas kernel here")


# speedup vs baseline: 2.3168x; 2.3168x over previous
import functools

import jax
import jax.numpy as jnp
from jax import lax
from jax.experimental import pallas as pl
from jax.experimental.pallas import tpu as pltpu
from jax.experimental.pallas import tpu_sc as plsc

E = 2049
EP = 2064
R = 448
NW = 32
RPW = 16
NACT = R // RPW
BM = 512


def _pad16(x, fill):
    n = x.shape[0]
    p = (-n) % 16
    return jnp.pad(x, (0, p), constant_values=fill), (n + p) // 16


def _densify_sc(vals_f, vals_t, rows_f, cols_f, rows_t, cols_t):
    rt, nt = _pad16(rows_t, -1)
    ct, _ = _pad16(cols_t, 0)
    vt, _ = _pad16(vals_t, 0.0)
    rf, nf = _pad16(rows_f, -1)
    cf, _ = _pad16(cols_f, 0)
    vf, _ = _pad16(vals_f, 0.0)

    mesh = plsc.VectorSubcoreMesh(core_axis_name="c", subcore_axis_name="s")

    slab = RPW * EP

    @functools.partial(
        pl.kernel,
        mesh=mesh,
        out_type=[jax.ShapeDtypeStruct((R * EP,), jnp.float32)] * 2,
        scratch_types=[
            pltpu.VMEM((nt * 16,), jnp.int32),
            pltpu.VMEM((nt * 16,), jnp.int32),
            pltpu.VMEM((nt * 16,), jnp.float32),
            pltpu.VMEM((nf * 16,), jnp.int32),
            pltpu.VMEM((nf * 16,), jnp.int32),
            pltpu.VMEM((nf * 16,), jnp.float32),
            pltpu.VMEM((slab,), jnp.float32),
            pltpu.VMEM((slab,), jnp.float32),
        ],
        compiler_params=pltpu.CompilerParams(needs_layout_passes=False),
    )
    def build(rt_h, ct_h, vt_h, rf_h, cf_h, vf_h, mt_h, mf_h,
              rt_v, ct_v, vt_v, rf_v, cf_v, vf_v, bt, bf):
        wid = lax.axis_index("s") * 2 + lax.axis_index("c")
        base = wid * RPW

        @pl.when(wid < NACT)
        def _():
            pltpu.sync_copy(rt_h, rt_v)
            pltpu.sync_copy(ct_h, ct_v)
            pltpu.sync_copy(vt_h, vt_v)
            pltpu.sync_copy(rf_h, rf_v)
            pltpu.sync_copy(cf_h, cf_v)
            pltpu.sync_copy(vf_h, vf_v)

            zeros16 = jnp.zeros((16,), jnp.float32)

            def zrow(j, c):
                bt[pl.ds(j * 16, 16)] = zeros16
                bf[pl.ds(j * 16, 16)] = zeros16
                return c
            lax.fori_loop(0, slab // 16, zrow, 0)

            def scat(k, c, rv, cv, vv, buf):
                i = k * 16
                r16 = rv[pl.ds(i, 16)]
                c16 = cv[pl.ds(i, 16)]
                v16 = vv[pl.ds(i, 16)]
                lr = r16 - base
                m = (lr >= 0) & (lr < RPW)
                lrc = jnp.clip(lr, 0, RPW - 1)
                plsc.store_scatter(buf, [lrc * EP + c16], v16, mask=m)
                return c

            lax.fori_loop(0, nt, functools.partial(scat, rv=rt_v, cv=ct_v, vv=vt_v, buf=bt), 0)
            lax.fori_loop(0, nf, functools.partial(scat, rv=rf_v, cv=cf_v, vv=vf_v, buf=bf), 0)

            pltpu.sync_copy(bt, mt_h.at[pl.ds(base * EP, slab)])
            pltpu.sync_copy(bf, mf_h.at[pl.ds(base * EP, slab)])

    mt_flat, mf_flat = build(rt, ct, vt, rf, cf, vf)
    return mt_flat.reshape(R, EP), mf_flat.reshape(R, EP)


def _mm_body(x_ref, s_ref, mt_ref, mf_ref, ot_ref, of_ref):
    a = x_ref[...]
    b = s_ref[...]
    mt = mt_ref[...][:, :E]
    mf = mf_ref[...][:, :E]
    dn = (((1,), (1,)), ((), ()))
    ot_ref[...] = lax.dot_general(a, mt, dn, precision=lax.Precision.HIGHEST,
                                  preferred_element_type=jnp.float32)
    of_ref[...] = lax.dot_general(b, mf, dn, precision=lax.Precision.HIGHEST,
                                  preferred_element_type=jnp.float32)


def kernel(ceps, spec, vals_f, vals_t, rows_f, cols_f, rows_t, cols_t):
    batch, steps, _ = ceps.shape
    n = batch * steps
    x = ceps.reshape(n, E)
    s = spec.reshape(n, E)

    mt, mf = _densify_sc(vals_f, vals_t, rows_f, cols_f, rows_t, cols_t)

    grid = (n // BM,)
    ppt, ppf = pl.pallas_call(
        _mm_body,
        grid=grid,
        in_specs=[
            pl.BlockSpec((BM, E), lambda i: (i, 0)),
            pl.BlockSpec((BM, E), lambda i: (i, 0)),
            pl.BlockSpec((R, EP), lambda i: (0, 0)),
            pl.BlockSpec((R, EP), lambda i: (0, 0)),
        ],
        out_specs=[
            pl.BlockSpec((BM, R), lambda i: (i, 0)),
            pl.BlockSpec((BM, R), lambda i: (i, 0)),
        ],
        out_shape=[jax.ShapeDtypeStruct((n, R), jnp.float32)] * 2,
    )(x, s, mt, mf)

    return ppt.reshape(batch, steps, R), ppf.reshape(batch, steps, R)


# speedup vs baseline: 2.4619x; 1.0626x over previous
import functools

import jax
import jax.numpy as jnp
from jax import lax
from jax.experimental import pallas as pl
from jax.experimental.pallas import tpu as pltpu
from jax.experimental.pallas import tpu_sc as plsc

E = 2049
EP = 2064
R = 448
RPW = 16
NACT = R // RPW


def _pad16(x, fill):
    n = x.shape[0]
    p = (-n) % 16
    return jnp.pad(x, (0, p), constant_values=fill), (n + p) // 16


def _densify_sc(vals_f, vals_t, rows_f, cols_f, rows_t, cols_t):
    rt, nt = _pad16(rows_t, -1)
    ct, _ = _pad16(cols_t, 0)
    vt, _ = _pad16(vals_t, 0.0)
    rf, nf = _pad16(rows_f, -1)
    cf, _ = _pad16(cols_f, 0)
    vf, _ = _pad16(vals_f, 0.0)

    mesh = plsc.VectorSubcoreMesh(core_axis_name="c", subcore_axis_name="s")

    @functools.partial(
        pl.kernel,
        mesh=mesh,
        out_type=[jax.ShapeDtypeStruct((R, EP), jnp.float32)] * 2,
        scratch_types=[
            pltpu.VMEM((nt * 16,), jnp.int32),
            pltpu.VMEM((nt * 16,), jnp.int32),
            pltpu.VMEM((nt * 16,), jnp.float32),
            pltpu.VMEM((nf * 16,), jnp.int32),
            pltpu.VMEM((nf * 16,), jnp.int32),
            pltpu.VMEM((nf * 16,), jnp.float32),
            pltpu.VMEM((RPW, EP), jnp.float32),
            pltpu.VMEM((RPW, EP), jnp.float32),
        ],
        compiler_params=pltpu.CompilerParams(needs_layout_passes=False),
    )
    def build(rt_h, ct_h, vt_h, rf_h, cf_h, vf_h, mt_h, mf_h,
              rt_v, ct_v, vt_v, rf_v, cf_v, vf_v, bt, bf):
        wid = lax.axis_index("s") * 2 + lax.axis_index("c")
        base = wid * RPW

        @pl.when(wid < NACT)
        def _():
            pltpu.sync_copy(rt_h, rt_v)
            pltpu.sync_copy(ct_h, ct_v)
            pltpu.sync_copy(vt_h, vt_v)
            pltpu.sync_copy(rf_h, rf_v)
            pltpu.sync_copy(cf_h, cf_v)
            pltpu.sync_copy(vf_h, vf_v)

            zeros16 = jnp.zeros((16,), jnp.float32)
            for r in range(RPW):
                def zrow(j, c, r=r):
                    bt[r, pl.ds(j * 16, 16)] = zeros16
                    bf[r, pl.ds(j * 16, 16)] = zeros16
                    return c
                lax.fori_loop(0, EP // 16, zrow, 0)

            def scat(k, c, rv, cv, vv, buf):
                i = k * 16
                r16 = rv[pl.ds(i, 16)]
                c16 = cv[pl.ds(i, 16)]
                v16 = vv[pl.ds(i, 16)]
                lr = r16 - base
                m = (lr >= 0) & (lr < RPW)
                lrc = jnp.clip(lr, 0, RPW - 1)
                plsc.store_scatter(buf, [lrc, c16], v16, mask=m)
                return c

            lax.fori_loop(0, nt, functools.partial(scat, rv=rt_v, cv=ct_v, vv=vt_v, buf=bt), 0)
            lax.fori_loop(0, nf, functools.partial(scat, rv=rf_v, cv=cf_v, vv=vf_v, buf=bf), 0)

            pltpu.sync_copy(bt, mt_h.at[pl.ds(base, RPW), :])
            pltpu.sync_copy(bf, mf_h.at[pl.ds(base, RPW), :])

    return build(rt, ct, vt, rf, cf, vf)


def _mm_body(x_ref, s_ref, mt_ref, mf_ref, ot_ref, of_ref):
    a = x_ref[0]
    b = s_ref[0]
    mt = mt_ref[...][:, :E]
    mf = mf_ref[...][:, :E]
    dn = (((1,), (1,)), ((), ()))
    ot = lax.dot_general(a, mt, dn, precision=lax.Precision.HIGHEST,
                         preferred_element_type=jnp.float32)
    of = lax.dot_general(b, mf, dn, precision=lax.Precision.HIGHEST,
                         preferred_element_type=jnp.float32)
    ot_ref[...] = ot[None]
    of_ref[...] = of[None]


def kernel(ceps, spec, vals_f, vals_t, rows_f, cols_f, rows_t, cols_t):
    batch, steps, _ = ceps.shape

    mt, mf = _densify_sc(vals_f, vals_t, rows_f, cols_f, rows_t, cols_t)

    ppt, ppf = pl.pallas_call(
        _mm_body,
        grid=(batch,),
        in_specs=[
            pl.BlockSpec((1, steps, E), lambda i: (i, 0, 0)),
            pl.BlockSpec((1, steps, E), lambda i: (i, 0, 0)),
            pl.BlockSpec((R, EP), lambda i: (0, 0)),
            pl.BlockSpec((R, EP), lambda i: (0, 0)),
        ],
        out_specs=[
            pl.BlockSpec((1, steps, R), lambda i: (i, 0, 0)),
            pl.BlockSpec((1, steps, R), lambda i: (i, 0, 0)),
        ],
        out_shape=[jax.ShapeDtypeStruct((batch, steps, R), jnp.float32)] * 2,
    )(ceps, spec, mt, mf)

    return ppt, ppf


# speedup vs baseline: 3.9500x; 1.6044x over previous
import functools

import jax
import jax.numpy as jnp
from jax import lax
from jax.experimental import pallas as pl
from jax.experimental.pallas import tpu as pltpu
from jax.experimental.pallas import tpu_sc as plsc

E = 2049
EP = 2064
R = 448
RPW = 16
NACT = R // RPW


def _pad16(x, fill):
    n = x.shape[0]
    p = (-n) % 16
    return jnp.pad(x, (0, p), constant_values=fill), (n + p) // 16


def _densify_sc(vals_f, vals_t, rows_f, cols_f, rows_t, cols_t):
    rt, nt = _pad16(rows_t, -1)
    ct, _ = _pad16(cols_t, 0)
    vt, _ = _pad16(vals_t, 0.0)
    rf, nf = _pad16(rows_f, -1)
    cf, _ = _pad16(cols_f, 0)
    vf, _ = _pad16(vals_f, 0.0)

    mesh = plsc.VectorSubcoreMesh(core_axis_name="c", subcore_axis_name="s")

    @functools.partial(
        pl.kernel,
        mesh=mesh,
        out_type=[jax.ShapeDtypeStruct((R, EP), jnp.float32)] * 2,
        scratch_types=[
            pltpu.VMEM((nt * 16,), jnp.int32),
            pltpu.VMEM((nt * 16,), jnp.int32),
            pltpu.VMEM((nt * 16,), jnp.float32),
            pltpu.VMEM((nf * 16,), jnp.int32),
            pltpu.VMEM((nf * 16,), jnp.int32),
            pltpu.VMEM((nf * 16,), jnp.float32),
            pltpu.VMEM((RPW, EP), jnp.float32),
            pltpu.VMEM((RPW, EP), jnp.float32),
        ],
        compiler_params=pltpu.CompilerParams(needs_layout_passes=False),
    )
    def build(rt_h, ct_h, vt_h, rf_h, cf_h, vf_h, mt_h, mf_h,
              rt_v, ct_v, vt_v, rf_v, cf_v, vf_v, bt, bf):
        wid = lax.axis_index("s") * 2 + lax.axis_index("c")
        base = wid * RPW

        @pl.when(wid < NACT)
        def _():
            pltpu.sync_copy(rt_h, rt_v)
            pltpu.sync_copy(ct_h, ct_v)
            pltpu.sync_copy(vt_h, vt_v)
            pltpu.sync_copy(rf_h, rf_v)
            pltpu.sync_copy(cf_h, cf_v)
            pltpu.sync_copy(vf_h, vf_v)

            zeros16 = jnp.zeros((16,), jnp.float32)
            for r in range(RPW):
                def zrow(j, c, r=r):
                    bt[r, pl.ds(j * 16, 16)] = zeros16
                    bf[r, pl.ds(j * 16, 16)] = zeros16
                    return c
                lax.fori_loop(0, EP // 16, zrow, 0)

            def scat(k, c, rv, cv, vv, buf):
                i = k * 16
                r16 = rv[pl.ds(i, 16)]
                c16 = cv[pl.ds(i, 16)]
                v16 = vv[pl.ds(i, 16)]
                lr = r16 - base
                m = (lr >= 0) & (lr < RPW)
                lrc = jnp.clip(lr, 0, RPW - 1)
                plsc.store_scatter(buf, [lrc, c16], v16, mask=m)
                return c

            lax.fori_loop(0, nt, functools.partial(scat, rv=rt_v, cv=ct_v, vv=vt_v, buf=bt), 0)
            lax.fori_loop(0, nf, functools.partial(scat, rv=rf_v, cv=cf_v, vv=vf_v, buf=bf), 0)

            pltpu.sync_copy(bt, mt_h.at[pl.ds(base, RPW), :])
            pltpu.sync_copy(bf, mf_h.at[pl.ds(base, RPW), :])

    return build(rt, ct, vt, rf, cf, vf)


def _mm_body(x_ref, s_ref, mt_ref, mf_ref, ot_ref, of_ref):
    a = x_ref[0].astype(jnp.bfloat16)
    b = s_ref[0].astype(jnp.bfloat16)
    mt = mt_ref[...][:, :E].astype(jnp.bfloat16)
    mf = mf_ref[...][:, :E].astype(jnp.bfloat16)
    dn = (((1,), (1,)), ((), ()))
    ot = lax.dot_general(a, mt, dn, precision=lax.Precision.DEFAULT,
                         preferred_element_type=jnp.float32)
    of = lax.dot_general(b, mf, dn, precision=lax.Precision.DEFAULT,
                         preferred_element_type=jnp.float32)
    ot_ref[...] = ot[None]
    of_ref[...] = of[None]


def kernel(ceps, spec, vals_f, vals_t, rows_f, cols_f, rows_t, cols_t):
    batch, steps, _ = ceps.shape

    mt, mf = _densify_sc(vals_f, vals_t, rows_f, cols_f, rows_t, cols_t)

    ppt, ppf = pl.pallas_call(
        _mm_body,
        grid=(batch,),
        in_specs=[
            pl.BlockSpec((1, steps, E), lambda i: (i, 0, 0)),
            pl.BlockSpec((1, steps, E), lambda i: (i, 0, 0)),
            pl.BlockSpec((R, EP), lambda i: (0, 0)),
            pl.BlockSpec((R, EP), lambda i: (0, 0)),
        ],
        out_specs=[
            pl.BlockSpec((1, steps, R), lambda i: (i, 0, 0)),
            pl.BlockSpec((1, steps, R), lambda i: (i, 0, 0)),
        ],
        out_shape=[jax.ShapeDtypeStruct((batch, steps, R), jnp.float32)] * 2,
    )(ceps, spec, mt, mf)

    return ppt, ppf
